# Initial kernel scaffold; baseline (speedup 1.0000x reference)
import functools

import jax
import jax.numpy as jnp
from jax import lax
from jax.experimental import pallas as pl
from jax.experimental.pallas import tpu as pltpu
from jax.experimental.pallas import tpu_sc as plsc

N = 10000
E = 320000
D_NODE = 128
D_EDGE = 16
H = 64

NC = 2
NS = 16
NW = NC * NS
EPW = E // NW
CH = 80
NCH = EPW // CH
RPT = N // NS
ZR = 125

f32 = jnp.float32
i32 = jnp.int32

_sc_mesh = plsc.VectorSubcoreMesh(core_axis_name="c", subcore_axis_name="s")


@functools.partial(
    pl.kernel,
    out_type=jax.ShapeDtypeStruct((E,), f32),
    mesh=_sc_mesh,
    scratch_types=[
        pltpu.VMEM((N,), f32),
        pltpu.VMEM((N,), f32),
        pltpu.VMEM((N,), f32),
        pltpu.VMEM((CH,), i32),
        pltpu.VMEM((CH,), i32),
        pltpu.VMEM((CH,), f32),
    ],
)
def _dij_kernel(px_hbm, py_hbm, pz_hbm, src_hbm, dst_hbm, dij_hbm,
                px_v, py_v, pz_v, si_v, di_v, o_v):
    core = lax.axis_index("c")
    sub = lax.axis_index("s")
    wid = sub * NC + core
    pltpu.sync_copy(px_hbm, px_v)
    pltpu.sync_copy(py_hbm, py_v)
    pltpu.sync_copy(pz_hbm, pz_v)
    base = wid * EPW

    def chunk(c, carry):
        eb = base + c * CH
        pltpu.sync_copy(src_hbm.at[pl.ds(eb, CH)], si_v)
        pltpu.sync_copy(dst_hbm.at[pl.ds(eb, CH)], di_v)

        def vec(i, carry2):
            s = si_v[pl.ds(i * 16, 16)]
            d = di_v[pl.ds(i * 16, 16)]
            dx = plsc.load_gather(px_v, [s]) - plsc.load_gather(px_v, [d])
            dy = plsc.load_gather(py_v, [s]) - plsc.load_gather(py_v, [d])
            dz = plsc.load_gather(pz_v, [s]) - plsc.load_gather(pz_v, [d])
            d2 = jnp.maximum(dx * dx + dy * dy + dz * dz, 1e-8)
            t = plsc.bitcast(d2, i32)
            t = 0x5F3759DF - lax.shift_right_logical(t, 1)
            y = plsc.bitcast(t, f32)
            for _ in range(3):
                y = y * (1.5 - 0.5 * d2 * y * y)
            o_v[pl.ds(i * 16, 16)] = d2 * y
            return carry2

        lax.fori_loop(0, CH // 16, vec, 0)
        pltpu.sync_copy(o_v, dij_hbm.at[pl.ds(eb, CH)])
        return carry

    lax.fori_loop(0, NCH, chunk, 0)


@functools.partial(
    pl.kernel,
    out_type=jax.ShapeDtypeStruct((NC, N, H), f32),
    mesh=_sc_mesh,
    scratch_types=[
        pltpu.VMEM((CH,), i32),
        pltpu.VMEM((CH,), i32),
        pltpu.VMEM((CH, H), f32),
        pltpu.VMEM((CH, H), f32),
        pltpu.VMEM((CH, H), f32),
        pltpu.VMEM((ZR, H), f32),
        pltpu.VMEM((RPT, H), f32),
        pltpu.SemaphoreType.DMA,
        pltpu.SemaphoreType.DMA,
        pltpu.VMEM_SHARED((N, H), f32),
    ],
)
def _edge_kernel(a_hbm, b_hbm, c_hbm, src_hbm, dst_hbm, out_hbm,
                 si_v, di_v, a_v, b_v, m_v, z_v, st_v, sem_a, sem_b, acc_sh):
    core = lax.axis_index("c")
    sub = lax.axis_index("s")
    wid = sub * NC + core
    row0 = sub * RPT

    def zrow(r, carry):
        for j in range(H // 16):
            z_v[r, pl.ds(j * 16, 16)] = jnp.zeros((16,), f32)
        return carry

    lax.fori_loop(0, ZR, zrow, 0)
    for k in range(RPT // ZR):
        pltpu.sync_copy(z_v, acc_sh.at[pl.ds(row0 + k * ZR, ZR)])
    plsc.subcore_barrier()

    base = wid * EPW

    def chunk(c, carry):
        eb = base + c * CH
        pltpu.sync_copy(src_hbm.at[pl.ds(eb, CH)], si_v)
        pltpu.sync_copy(dst_hbm.at[pl.ds(eb, CH)], di_v)
        cp_a = pltpu.async_copy(a_hbm.at[si_v], a_v, sem_a)
        cp_b = pltpu.async_copy(b_hbm.at[di_v], b_v, sem_b)
        pltpu.sync_copy(c_hbm.at[pl.ds(eb, CH)], m_v)
        cp_a.wait()
        cp_b.wait()

        def edge(e, carry2):
            for j in range(H // 16):
                sl = pl.ds(j * 16, 16)
                x = a_v[e, sl] + b_v[e, sl] + m_v[e, sl]
                m_v[e, sl] = x / (1.0 + jnp.exp(-x))
            return carry2

        lax.fori_loop(0, CH, edge, 0)
        pltpu.sync_copy(m_v, acc_sh.at[di_v], add=True)
        return carry

    lax.fori_loop(0, NCH, chunk, 0)
    plsc.subcore_barrier()
    pltpu.sync_copy(acc_sh.at[pl.ds(row0, RPT)], st_v)
    pltpu.sync_copy(st_v, out_hbm.at[core, pl.ds(row0, RPT)])


BE = 4000


def _cbase_body(d_ref, ea_ref, we_ref, wdb_ref, c0_ref, c1_ref, c2_ref):
    d = d_ref[...]
    ea = ea_ref[...]
    for l, c_ref in enumerate((c0_ref, c1_ref, c2_ref)):
        y = jnp.dot(ea, we_ref[l], preferred_element_type=f32)
        c_ref[...] = y + d * wdb_ref[l, 0:1, :] + wdb_ref[l, 1:2, :]


def _cbase(dij, ea, we, wdb):
    return pl.pallas_call(
        _cbase_body,
        grid=(E // BE,),
        in_specs=[
            pl.BlockSpec((BE, 1), lambda i: (i, 0)),
            pl.BlockSpec((BE, D_EDGE), lambda i: (i, 0)),
            pl.BlockSpec((3, D_EDGE, H), lambda i: (0, 0, 0)),
            pl.BlockSpec((3, 8, H), lambda i: (0, 0, 0)),
        ],
        out_specs=[pl.BlockSpec((BE, H), lambda i: (i, 0))] * 3,
        out_shape=[jax.ShapeDtypeStruct((E, H), f32)] * 3,
    )(dij, ea, we, wdb)


def _proj_body(h_ref, ws_ref, wd_ref, a_ref, b_ref):
    h = h_ref[...]
    a_ref[...] = jnp.dot(h, ws_ref[...], preferred_element_type=f32)
    b_ref[...] = jnp.dot(h, wd_ref[...], preferred_element_type=f32)


def _proj(h, ws, wd):
    return pl.pallas_call(
        _proj_body,
        out_shape=[jax.ShapeDtypeStruct((N, H), f32)] * 2,
    )(h, ws, wd)


def _update(mg_ref, uw_ref, vec_ref):
    m = (mg_ref[0] + mg_ref[1]) * 0.1
    u = jnp.dot(m, uw_ref[...], preferred_element_type=f32) + vec_ref[0:1]
    u = u * jax.nn.sigmoid(u)
    mu = jnp.mean(u, axis=1, keepdims=True)
    var = jnp.mean((u - mu) ** 2, axis=1, keepdims=True)
    return (u - mu) * lax.rsqrt(var + 1e-5) * vec_ref[1:2] + vec_ref[2:3]


def _upd_proj_body(mg_ref, uw_ref, vec_ref, ws_ref, wd_ref, a_ref, b_ref):
    hh = _update(mg_ref, uw_ref, vec_ref)
    a_ref[...] = jnp.dot(hh, ws_ref[...], preferred_element_type=f32)
    b_ref[...] = jnp.dot(hh, wd_ref[...], preferred_element_type=f32)


def _upd_proj(mg, uw, vec, ws, wd):
    return pl.pallas_call(
        _upd_proj_body,
        out_shape=[jax.ShapeDtypeStruct((N, H), f32)] * 2,
    )(mg, uw, vec, ws, wd)


def _upd_head_body(mg_ref, uw_ref, vec_ref, w1_ref, b1_ref, w2_ref, b2_ref,
                   o_ref):
    hh = _update(mg_ref, uw_ref, vec_ref)
    gf = jnp.sum(hh, axis=0, keepdims=True) * 0.1
    hid = jnp.dot(gf, w1_ref[...], preferred_element_type=f32) + b1_ref[...]
    hid = hid * jax.nn.sigmoid(hid)
    o_ref[...] = jnp.dot(hid, w2_ref[...], preferred_element_type=f32) + b2_ref[...]


def _upd_head(mg, uw, vec, w1, b1, w2, b2):
    return pl.pallas_call(
        _upd_head_body,
        out_shape=jax.ShapeDtypeStruct((1, 1), f32),
    )(mg, uw, vec, w1, b1, w2, b2)


def _vec3(ub, g, b):
    v = jnp.zeros((8, H), f32)
    return v.at[0].set(ub).at[1].set(g).at[2].set(b)


def kernel(x, pos, edge_index, edge_attr,
           msg_W0, msg_b0, upd_W0, upd_b0, ln_g0, ln_b0,
           msg_W1, msg_b1, upd_W1, upd_b1, ln_g1, ln_b1,
           msg_W2, msg_b2, upd_W2, upd_b2, ln_g2, ln_b2,
           head_W1, head_b1, head_W2, head_b2):
    src = edge_index[0]
    dst = edge_index[1]
    px, py, pz = pos[:, 0], pos[:, 1], pos[:, 2]

    dij = _dij_kernel(px, py, pz, src, dst)

    we_all, wdb_all = [], []
    for mW, mb, din in ((msg_W0, msg_b0, D_NODE), (msg_W1, msg_b1, H),
                        (msg_W2, msg_b2, H)):
        we_all.append(mW[2 * din + 1:])
        wdb = jnp.zeros((8, H), f32).at[0].set(mW[2 * din]).at[1].set(mb)
        wdb_all.append(wdb)
    c0, c1, c2 = _cbase(dij[:, None], edge_attr,
                        jnp.stack(we_all), jnp.stack(wdb_all))

    a, b = _proj(x, msg_W0[:D_NODE], msg_W0[D_NODE:2 * D_NODE])
    mg = _edge_kernel(a, b, c0, src, dst)
    a, b = _upd_proj(mg, upd_W0, _vec3(upd_b0, ln_g0, ln_b0),
                     msg_W1[:H], msg_W1[H:2 * H])
    mg = _edge_kernel(a, b, c1, src, dst)
    a, b = _upd_proj(mg, upd_W1, _vec3(upd_b1, ln_g1, ln_b1),
                     msg_W2[:H], msg_W2[H:2 * H])
    mg = _edge_kernel(a, b, c2, src, dst)
    return _upd_head(mg, upd_W2, _vec3(upd_b2, ln_g2, ln_b2),
                     head_W1, head_b1[None], head_W2, head_b2[None])


# speedup vs baseline: 3.6266x; 3.6266x over previous
import functools

import jax
import jax.numpy as jnp
from jax import lax
from jax.experimental import pallas as pl
from jax.experimental.pallas import tpu as pltpu
from jax.experimental.pallas import tpu_sc as plsc

N = 10000
E = 320000
D_NODE = 128
D_EDGE = 16
H = 64

NC = 2
NS = 16
NW = NC * NS
EPW = E // NW
CH = 80
NCH = EPW // CH
NP = 10240
RPT = NP // NS
ZR = 128

f32 = jnp.float32
i32 = jnp.int32

_sc_mesh = plsc.VectorSubcoreMesh(core_axis_name="c", subcore_axis_name="s",
                                  num_cores=NC, num_subcores=NS)
_sc_params = pltpu.CompilerParams(needs_layout_passes=False,
                                  use_tc_tiling_on_sc=False)


_dij_types = dict(
    out_type=jax.ShapeDtypeStruct((E,), f32),
    scratch_types=[
        pltpu.VMEM((N,), f32),
        pltpu.VMEM((N,), f32),
        pltpu.VMEM((N,), f32),
        pltpu.VMEM((CH,), i32),
        pltpu.VMEM((CH,), i32),
        pltpu.VMEM((CH,), f32),
    ],
)


def _dij_body(px_hbm, py_hbm, pz_hbm, src_hbm, dst_hbm, dij_hbm,
              px_v, py_v, pz_v, si_v, di_v, o_v):
    core = lax.axis_index("c")
    sub = lax.axis_index("s")
    wid = sub * NC + core
    pltpu.sync_copy(px_hbm, px_v)
    pltpu.sync_copy(py_hbm, py_v)
    pltpu.sync_copy(pz_hbm, pz_v)
    base = wid * EPW

    def chunk(c, carry):
        eb = base + c * CH
        pltpu.sync_copy(src_hbm.at[pl.ds(eb, CH)], si_v)
        pltpu.sync_copy(dst_hbm.at[pl.ds(eb, CH)], di_v)

        def vec(i, carry2):
            s = si_v[pl.ds(i * 16, 16)]
            d = di_v[pl.ds(i * 16, 16)]
            dx = plsc.load_gather(px_v, [s]) - plsc.load_gather(px_v, [d])
            dy = plsc.load_gather(py_v, [s]) - plsc.load_gather(py_v, [d])
            dz = plsc.load_gather(pz_v, [s]) - plsc.load_gather(pz_v, [d])
            d2 = jnp.maximum(dx * dx + dy * dy + dz * dz, 1e-8)
            t = plsc.bitcast(d2, i32)
            t = 0x5F3759DF - lax.shift_right_logical(t, 1)
            y = plsc.bitcast(t, f32)
            for _ in range(3):
                y = y * (1.5 - 0.5 * d2 * y * y)
            o_v[pl.ds(i * 16, 16)] = d2 * y
            return carry2

        lax.fori_loop(0, CH // 16, vec, 0)
        pltpu.sync_copy(o_v, dij_hbm.at[pl.ds(eb, CH)])
        return carry

    lax.fori_loop(0, NCH, chunk, 0)


_dij_kernel = pl.kernel(_dij_body, mesh=_sc_mesh, compiler_params=_sc_params,
                        **_dij_types)


_edge_types = dict(
    out_type=jax.ShapeDtypeStruct((NC, NP, H), f32),
    scratch_types=[
        pltpu.VMEM((CH,), i32),
        pltpu.VMEM((CH,), i32),
        pltpu.VMEM((CH, H), f32),
        pltpu.VMEM((CH, H), f32),
        pltpu.VMEM((CH, H), f32),
        pltpu.VMEM((ZR, H), f32),
        pltpu.VMEM((RPT, H), f32),
        pltpu.SemaphoreType.DMA,
        pltpu.SemaphoreType.DMA,
        pltpu.VMEM_SHARED((NP, H), f32),
    ],
)


def _edge_body(a_hbm, b_hbm, c_hbm, src_hbm, dst_hbm, out_hbm,
               si_v, di_v, a_v, b_v, m_v, z_v, st_v, sem_a, sem_b, acc_sh):
    core = lax.axis_index("c")
    sub = lax.axis_index("s")
    wid = sub * NC + core
    row0 = sub * RPT

    def zrow(r, carry):
        for j in range(H // 16):
            z_v[r, pl.ds(j * 16, 16)] = jnp.zeros((16,), f32)
        return carry

    lax.fori_loop(0, ZR, zrow, 0)
    for k in range(RPT // ZR):
        pltpu.sync_copy(z_v, acc_sh.at[pl.ds(row0 + k * ZR, ZR)])
    plsc.subcore_barrier()

    base = wid * EPW

    def chunk(c, carry):
        eb = base + c * CH
        pltpu.sync_copy(src_hbm.at[pl.ds(eb, CH)], si_v)
        pltpu.sync_copy(dst_hbm.at[pl.ds(eb, CH)], di_v)
        cp_a = pltpu.async_copy(a_hbm.at[si_v], a_v, sem_a)
        cp_b = pltpu.async_copy(b_hbm.at[di_v], b_v, sem_b)
        pltpu.sync_copy(c_hbm.at[pl.ds(eb, CH)], m_v)
        cp_a.wait()
        cp_b.wait()

        def edge(e, carry2):
            for j in range(H // 16):
                sl = pl.ds(j * 16, 16)
                x = a_v[e, sl] + b_v[e, sl] + m_v[e, sl]
                m_v[e, sl] = x / (1.0 + jnp.exp(-x))
            return carry2

        lax.fori_loop(0, CH, edge, 0)
        pltpu.sync_copy(m_v, acc_sh.at[di_v], add=True)
        return carry

    lax.fori_loop(0, NCH, chunk, 0)
    plsc.subcore_barrier()
    pltpu.sync_copy(acc_sh.at[pl.ds(row0, RPT)], st_v)
    pltpu.sync_copy(st_v, out_hbm.at[core, pl.ds(row0, RPT)])


_edge_kernel = pl.kernel(_edge_body, mesh=_sc_mesh, compiler_params=_sc_params,
                         **_edge_types)


BE = 4000


def _cbase_body(d_ref, ea_ref, we_ref, wdb_ref, c0_ref, c1_ref, c2_ref):
    d = d_ref[...]
    ea = ea_ref[...]
    for l, c_ref in enumerate((c0_ref, c1_ref, c2_ref)):
        y = jnp.dot(ea, we_ref[l], preferred_element_type=f32)
        c_ref[...] = y + d * wdb_ref[l, 0:1, :] + wdb_ref[l, 1:2, :]


def _cbase(dij, ea, we, wdb):
    return pl.pallas_call(
        _cbase_body,
        grid=(E // BE,),
        in_specs=[
            pl.BlockSpec((BE, 1), lambda i: (i, 0)),
            pl.BlockSpec((BE, D_EDGE), lambda i: (i, 0)),
            pl.BlockSpec((3, D_EDGE, H), lambda i: (0, 0, 0)),
            pl.BlockSpec((3, 8, H), lambda i: (0, 0, 0)),
        ],
        out_specs=[pl.BlockSpec((BE, H), lambda i: (i, 0))] * 3,
        out_shape=[jax.ShapeDtypeStruct((E, H), f32)] * 3,
    )(dij, ea, we, wdb)


def _proj_body(h_ref, ws_ref, wd_ref, a_ref, b_ref):
    h = h_ref[...]
    a_ref[...] = jnp.dot(h, ws_ref[...], preferred_element_type=f32)
    b_ref[...] = jnp.dot(h, wd_ref[...], preferred_element_type=f32)


def _proj(h, ws, wd):
    return pl.pallas_call(
        _proj_body,
        out_shape=[jax.ShapeDtypeStruct((N, H), f32)] * 2,
    )(h, ws, wd)


def _update(mg_ref, uw_ref, vec_ref):
    m = (mg_ref[0, :N] + mg_ref[1, :N]) * 0.1
    u = jnp.dot(m, uw_ref[...], preferred_element_type=f32) + vec_ref[0:1]
    u = u * jax.nn.sigmoid(u)
    mu = jnp.mean(u, axis=1, keepdims=True)
    var = jnp.mean((u - mu) ** 2, axis=1, keepdims=True)
    return (u - mu) / jnp.sqrt(var + 1e-5) * vec_ref[1:2] + vec_ref[2:3]


def _upd_proj_body(mg_ref, uw_ref, vec_ref, ws_ref, wd_ref, a_ref, b_ref):
    hh = _update(mg_ref, uw_ref, vec_ref)
    a_ref[...] = jnp.dot(hh, ws_ref[...], preferred_element_type=f32)
    b_ref[...] = jnp.dot(hh, wd_ref[...], preferred_element_type=f32)


def _upd_proj(mg, uw, vec, ws, wd):
    return pl.pallas_call(
        _upd_proj_body,
        out_shape=[jax.ShapeDtypeStruct((N, H), f32)] * 2,
    )(mg, uw, vec, ws, wd)


def _upd_head_body(mg_ref, uw_ref, vec_ref, w1_ref, b1_ref, w2_ref, b2_ref,
                   o_ref):
    hh = _update(mg_ref, uw_ref, vec_ref)
    gf = jnp.sum(hh, axis=0, keepdims=True) * 0.1
    hid = jnp.dot(gf, w1_ref[...], preferred_element_type=f32) + b1_ref[...]
    hid = hid * jax.nn.sigmoid(hid)
    o_ref[...] = jnp.dot(hid, w2_ref[...], preferred_element_type=f32) + b2_ref[...]


def _upd_head(mg, uw, vec, w1, b1, w2, b2):
    return pl.pallas_call(
        _upd_head_body,
        out_shape=jax.ShapeDtypeStruct((1, 1), f32),
    )(mg, uw, vec, w1, b1, w2, b2)


def _vec3(ub, g, b):
    v = jnp.zeros((8, H), f32)
    return v.at[0].set(ub).at[1].set(g).at[2].set(b)


def kernel(x, pos, edge_index, edge_attr,
           msg_W0, msg_b0, upd_W0, upd_b0, ln_g0, ln_b0,
           msg_W1, msg_b1, upd_W1, upd_b1, ln_g1, ln_b1,
           msg_W2, msg_b2, upd_W2, upd_b2, ln_g2, ln_b2,
           head_W1, head_b1, head_W2, head_b2):
    src = edge_index[0]
    dst = edge_index[1]
    px, py, pz = pos[:, 0], pos[:, 1], pos[:, 2]

    dij = _dij_kernel(px, py, pz, src, dst)

    we_all, wdb_all = [], []
    for mW, mb, din in ((msg_W0, msg_b0, D_NODE), (msg_W1, msg_b1, H),
                        (msg_W2, msg_b2, H)):
        we_all.append(mW[2 * din + 1:])
        wdb = jnp.zeros((8, H), f32).at[0].set(mW[2 * din]).at[1].set(mb)
        wdb_all.append(wdb)
    c0, c1, c2 = _cbase(dij[:, None], edge_attr,
                        jnp.stack(we_all), jnp.stack(wdb_all))

    a, b = _proj(x, msg_W0[:D_NODE], msg_W0[D_NODE:2 * D_NODE])
    mg = _edge_kernel(a, b, c0, src, dst)
    a, b = _upd_proj(mg, upd_W0, _vec3(upd_b0, ln_g0, ln_b0),
                     msg_W1[:H], msg_W1[H:2 * H])
    mg = _edge_kernel(a, b, c1, src, dst)
    a, b = _upd_proj(mg, upd_W1, _vec3(upd_b1, ln_g1, ln_b1),
                     msg_W2[:H], msg_W2[H:2 * H])
    mg = _edge_kernel(a, b, c2, src, dst)
    return _upd_head(mg, upd_W2, _vec3(upd_b2, ln_g2, ln_b2),
                     head_W1, head_b1[None], head_W2, head_b2[None])


# speedup vs baseline: 6.7195x; 1.8529x over previous
import functools

import jax
import jax.numpy as jnp
from jax import lax
from jax.experimental import pallas as pl
from jax.experimental.pallas import tpu as pltpu
from jax.experimental.pallas import tpu_sc as plsc

N = 10000
E = 320000
D_NODE = 128
D_EDGE = 16
H = 64

NC = 2
NS = 16
NW = NC * NS
EPW = E // NW
CH = 80
NCH = EPW // CH
NP = 10240
RPT = NP // NS
ZR = 128

f32 = jnp.float32
i32 = jnp.int32

_sc_mesh = plsc.VectorSubcoreMesh(core_axis_name="c", subcore_axis_name="s",
                                  num_cores=NC, num_subcores=NS)
_sc_params = pltpu.CompilerParams(needs_layout_passes=False,
                                  use_tc_tiling_on_sc=False)


_dij_types = dict(
    out_type=jax.ShapeDtypeStruct((E,), f32),
    scratch_types=[
        pltpu.VMEM((N,), f32),
        pltpu.VMEM((N,), f32),
        pltpu.VMEM((N,), f32),
        pltpu.VMEM((EPW,), i32),
        pltpu.VMEM((EPW,), i32),
        pltpu.VMEM((EPW,), f32),
    ],
)


def _dij_body(px_hbm, py_hbm, pz_hbm, src_hbm, dst_hbm, dij_hbm,
              px_v, py_v, pz_v, si_v, di_v, o_v):
    core = lax.axis_index("c")
    sub = lax.axis_index("s")
    wid = sub * NC + core
    base = wid * EPW
    pltpu.sync_copy(px_hbm, px_v)
    pltpu.sync_copy(py_hbm, py_v)
    pltpu.sync_copy(pz_hbm, pz_v)
    pltpu.sync_copy(src_hbm.at[pl.ds(base, EPW)], si_v)
    pltpu.sync_copy(dst_hbm.at[pl.ds(base, EPW)], di_v)

    @plsc.parallel_loop(0, EPW // 16, 1, unroll=2)
    def _vec(i):
        s = si_v[pl.ds(i * 16, 16)]
        d = di_v[pl.ds(i * 16, 16)]
        dx = plsc.load_gather(px_v, [s]) - plsc.load_gather(px_v, [d])
        dy = plsc.load_gather(py_v, [s]) - plsc.load_gather(py_v, [d])
        dz = plsc.load_gather(pz_v, [s]) - plsc.load_gather(pz_v, [d])
        d2 = jnp.maximum(dx * dx + dy * dy + dz * dz, 1e-8)
        t = plsc.bitcast(d2, i32)
        t = 0x5F3759DF - lax.shift_right_logical(t, 1)
        y = plsc.bitcast(t, f32)
        for _ in range(3):
            y = y * (1.5 - 0.5 * d2 * y * y)
        o_v[pl.ds(i * 16, 16)] = d2 * y

    pltpu.sync_copy(o_v, dij_hbm.at[pl.ds(base, EPW)])


_dij_kernel = pl.kernel(_dij_body, mesh=_sc_mesh, compiler_params=_sc_params,
                        **_dij_types)


_edge_types = dict(
    out_type=jax.ShapeDtypeStruct((NC, NP, H), f32),
    scratch_types=[
        pltpu.VMEM((NCH, CH), i32),
        pltpu.VMEM((NCH, CH), i32),
        pltpu.VMEM((2, CH, H), f32),
        pltpu.VMEM((2, CH, H), f32),
        pltpu.VMEM((2, CH, H), f32),
        pltpu.VMEM((ZR, H), f32),
        [pltpu.SemaphoreType.DMA] * 2,
        [pltpu.SemaphoreType.DMA] * 2,
        [pltpu.SemaphoreType.DMA] * 2,
        pltpu.VMEM_SHARED((NP, H), f32),
    ],
)


def _edge_body(a_hbm, b_hbm, c_hbm, src3_hbm, dst3_hbm, out_hbm,
               si_v, di_v, a_v, b_v, m_v, z_v, sem_a, sem_b, sem_c,
               acc_sh):
    core = lax.axis_index("c")
    sub = lax.axis_index("s")
    wid = sub * NC + core
    row0 = sub * RPT
    base = wid * EPW

    pltpu.sync_copy(src3_hbm.at[wid], si_v)
    pltpu.sync_copy(dst3_hbm.at[wid], di_v)

    def zrow(r, carry):
        for j in range(H // 16):
            z_v[r, pl.ds(j * 16, 16)] = jnp.zeros((16,), f32)
        return carry

    lax.fori_loop(0, ZR, zrow, 0)
    for k in range(RPT // ZR):
        pltpu.sync_copy(z_v, acc_sh.at[pl.ds(row0 + k * ZR, ZR)])
    plsc.subcore_barrier()

    def gathers(c, par):
        return (
            pltpu.make_async_copy(a_hbm.at[si_v.at[c]], a_v.at[par], sem_a[par]),
            pltpu.make_async_copy(b_hbm.at[di_v.at[c]], b_v.at[par], sem_b[par]),
            pltpu.make_async_copy(c_hbm.at[pl.ds(base + c * CH, CH)],
                                  m_v.at[par], sem_c[par]),
        )

    def start(c, par):
        for cp in gathers(c, par):
            cp.start()

    def finish(c, par):
        for cp in gathers(c, par):
            cp.wait()

        @plsc.parallel_loop(0, CH, 1, unroll=2)
        def _edge(e):
            for j in range(H // 16):
                sl = pl.ds(j * 16, 16)
                x = a_v[par, e, sl] + b_v[par, e, sl] + m_v[par, e, sl]
                m_v[par, e, sl] = x / (1.0 + jnp.exp(-x))

        pltpu.sync_copy(m_v.at[par], acc_sh.at[di_v.at[c]], add=True)

    start(0, 0)

    def pair(p, carry):
        c0 = 2 * p
        start(c0 + 1, 1)
        finish(c0, 0)
        start(c0 + 2, 0)
        finish(c0 + 1, 1)
        return carry

    lax.fori_loop(0, (NCH - 1) // 2, pair, 0)
    finish(NCH - 1, 0)
    plsc.subcore_barrier()
    pltpu.sync_copy(acc_sh.at[pl.ds(row0, RPT)],
                    out_hbm.at[core, pl.ds(row0, RPT)])


_edge_kernel = pl.kernel(_edge_body, mesh=_sc_mesh, compiler_params=_sc_params,
                         **_edge_types)


BE = 4000


def _cbase_body(d_ref, ea_ref, we_ref, wdb_ref, c0_ref, c1_ref, c2_ref):
    d = d_ref[...]
    ea = ea_ref[...]
    for l, c_ref in enumerate((c0_ref, c1_ref, c2_ref)):
        y = jnp.dot(ea, we_ref[l], preferred_element_type=f32)
        c_ref[...] = y + d * wdb_ref[l, 0:1, :] + wdb_ref[l, 1:2, :]


def _cbase(dij, ea, we, wdb):
    return pl.pallas_call(
        _cbase_body,
        grid=(E // BE,),
        in_specs=[
            pl.BlockSpec((BE, 1), lambda i: (i, 0)),
            pl.BlockSpec((BE, D_EDGE), lambda i: (i, 0)),
            pl.BlockSpec((3, D_EDGE, H), lambda i: (0, 0, 0)),
            pl.BlockSpec((3, 8, H), lambda i: (0, 0, 0)),
        ],
        out_specs=[pl.BlockSpec((BE, H), lambda i: (i, 0))] * 3,
        out_shape=[jax.ShapeDtypeStruct((E, H), f32)] * 3,
    )(dij, ea, we, wdb)


def _proj_body(h_ref, ws_ref, wd_ref, a_ref, b_ref):
    h = h_ref[...]
    a_ref[...] = jnp.dot(h, ws_ref[...], preferred_element_type=f32)
    b_ref[...] = jnp.dot(h, wd_ref[...], preferred_element_type=f32)


def _proj(h, ws, wd):
    return pl.pallas_call(
        _proj_body,
        out_shape=[jax.ShapeDtypeStruct((N, H), f32)] * 2,
    )(h, ws, wd)


def _update(mg_ref, uw_ref, vec_ref):
    m = (mg_ref[0, :N] + mg_ref[1, :N]) * 0.1
    u = jnp.dot(m, uw_ref[...], preferred_element_type=f32) + vec_ref[0:1]
    u = u * jax.nn.sigmoid(u)
    mu = jnp.mean(u, axis=1, keepdims=True)
    var = jnp.mean((u - mu) ** 2, axis=1, keepdims=True)
    return (u - mu) / jnp.sqrt(var + 1e-5) * vec_ref[1:2] + vec_ref[2:3]


def _upd_proj_body(mg_ref, uw_ref, vec_ref, ws_ref, wd_ref, a_ref, b_ref):
    hh = _update(mg_ref, uw_ref, vec_ref)
    a_ref[...] = jnp.dot(hh, ws_ref[...], preferred_element_type=f32)
    b_ref[...] = jnp.dot(hh, wd_ref[...], preferred_element_type=f32)


def _upd_proj(mg, uw, vec, ws, wd):
    return pl.pallas_call(
        _upd_proj_body,
        out_shape=[jax.ShapeDtypeStruct((N, H), f32)] * 2,
    )(mg, uw, vec, ws, wd)


def _upd_head_body(mg_ref, uw_ref, vec_ref, w1_ref, b1_ref, w2_ref, b2_ref,
                   o_ref):
    hh = _update(mg_ref, uw_ref, vec_ref)
    gf = jnp.sum(hh, axis=0, keepdims=True) * 0.1
    hid = jnp.dot(gf, w1_ref[...], preferred_element_type=f32) + b1_ref[...]
    hid = hid * jax.nn.sigmoid(hid)
    o_ref[...] = jnp.dot(hid, w2_ref[...], preferred_element_type=f32) + b2_ref[...]


def _upd_head(mg, uw, vec, w1, b1, w2, b2):
    return pl.pallas_call(
        _upd_head_body,
        out_shape=jax.ShapeDtypeStruct((1, 1), f32),
    )(mg, uw, vec, w1, b1, w2, b2)


def _vec3(ub, g, b):
    v = jnp.zeros((8, H), f32)
    return v.at[0].set(ub).at[1].set(g).at[2].set(b)


def kernel(x, pos, edge_index, edge_attr,
           msg_W0, msg_b0, upd_W0, upd_b0, ln_g0, ln_b0,
           msg_W1, msg_b1, upd_W1, upd_b1, ln_g1, ln_b1,
           msg_W2, msg_b2, upd_W2, upd_b2, ln_g2, ln_b2,
           head_W1, head_b1, head_W2, head_b2):
    src = edge_index[0]
    dst = edge_index[1]
    src3 = src.reshape(NW, NCH, CH)
    dst3 = dst.reshape(NW, NCH, CH)
    px, py, pz = pos[:, 0], pos[:, 1], pos[:, 2]

    dij = _dij_kernel(px, py, pz, src, dst)

    we_all, wdb_all = [], []
    for mW, mb, din in ((msg_W0, msg_b0, D_NODE), (msg_W1, msg_b1, H),
                        (msg_W2, msg_b2, H)):
        we_all.append(mW[2 * din + 1:])
        wdb = jnp.zeros((8, H), f32).at[0].set(mW[2 * din]).at[1].set(mb)
        wdb_all.append(wdb)
    c0, c1, c2 = _cbase(dij[:, None], edge_attr,
                        jnp.stack(we_all), jnp.stack(wdb_all))

    a, b = _proj(x, msg_W0[:D_NODE], msg_W0[D_NODE:2 * D_NODE])
    mg = _edge_kernel(a, b, c0, src3, dst3)
    a, b = _upd_proj(mg, upd_W0, _vec3(upd_b0, ln_g0, ln_b0),
                     msg_W1[:H], msg_W1[H:2 * H])
    mg = _edge_kernel(a, b, c1, src3, dst3)
    a, b = _upd_proj(mg, upd_W1, _vec3(upd_b1, ln_g1, ln_b1),
                     msg_W2[:H], msg_W2[H:2 * H])
    mg = _edge_kernel(a, b, c2, src3, dst3)
    return _upd_head(mg, upd_W2, _vec3(upd_b2, ln_g2, ln_b2),
                     head_W1, head_b1[None], head_W2, head_b2[None])
